# baseline (speedup 1.0000x reference)
import functools

import jax
import jax.numpy as jnp
from jax import lax
from jax.experimental import pallas as pl
from jax.experimental.pallas import tpu as pltpu
from jax.experimental.pallas import tpu_sc as plsc

_L = 16
_NS = 16
_ROWS = 2048
_K = 8
_V = 100000
_E = _ROWS * _K
_EPT = _E // _NS
_CH = 128
_NDMA = _EPT // _CH
_RPT = _ROWS // _NS


def _sc_body(pred_hbm, idx_hbm, st_hbm, mask_hbm, out_hbm,
             idx_v, vals_v, st_v, mask_v, red_v, sem):
    wid = lax.axis_index("s")
    lane = lax.iota(jnp.int32, _L)
    sub = lane >> 3

    pltpu.sync_copy(idx_hbm.at[pl.ds(wid * _NDMA, _NDMA)], idx_v)
    pltpu.sync_copy(st_hbm.at[pl.ds(wid * _NDMA, _NDMA)], st_v)
    pltpu.sync_copy(mask_hbm.at[wid], mask_v)

    base_e = wid * _EPT
    for j in range(_NDMA):
        for k in range(_CH // _L):
            seg = pl.ds(k * _L, _L)
            row = (base_e + j * _CH + k * _L) // _K + sub
            idx_v[j, seg] = idx_v[j, seg] + row * _V

    handles = [
        pltpu.make_async_copy(pred_hbm.at[idx_v.at[j]], vals_v.at[j], sem)
        for j in range(_NDMA)
    ]
    for h in handles:
        h.start()
    for h in handles:
        h.wait()

    acc = jnp.zeros((_L,), jnp.float32)
    for j in range(_NDMA):
        mrow = mask_v[pl.ds(j * _L, _L)]
        for k in range(_CH // _L):
            seg = pl.ds(k * _L, _L)
            m0 = jnp.full((_L,), mrow[2 * k], jnp.float32)
            m1 = jnp.full((_L,), mrow[2 * k + 1], jnp.float32)
            m = jnp.where(sub == 0, m0, m1)
            acc = acc + vals_v[j, seg] * st_v[j, seg] * m

    macc = jnp.zeros((_L,), jnp.float32)
    for k in range(_RPT // _L):
        macc = macc + mask_v[pl.ds(k * _L, _L)]

    s_t = jnp.sum(acc)
    m_t = jnp.sum(macc)
    partial = jnp.where(lane == 0, jnp.full((_L,), s_t, jnp.float32),
                        jnp.where(lane == 1, jnp.full((_L,), m_t, jnp.float32),
                                  jnp.zeros((_L,), jnp.float32)))
    red_v[...] = partial
    pltpu.sync_copy(red_v, out_hbm.at[wid])


def _combine_body(p_ref, o_ref):
    p = p_ref[...]
    s = jnp.sum(p[:, 0])
    m = jnp.sum(p[:, 1])
    o_ref[...] = jnp.full((1, 1), -(s / m), jnp.float32)


def kernel(pred, idxs, soft_target, mask):
    pred_flat = pred.reshape(_ROWS * _V)
    idx2 = idxs.astype(jnp.int32).reshape(_E // _CH, _CH)
    st2 = soft_target.astype(jnp.float32).reshape(_E // _CH, _CH)
    mask2 = mask.astype(jnp.float32).reshape(_NS, _RPT)

    mesh = plsc.VectorSubcoreMesh(
        core_axis_name="c", subcore_axis_name="s", num_cores=1)
    run = functools.partial(
        pl.kernel,
        mesh=mesh,
        compiler_params=pltpu.CompilerParams(needs_layout_passes=False),
        out_type=jax.ShapeDtypeStruct((_NS, _L), jnp.float32),
        scratch_types=[
            pltpu.VMEM((_NDMA, _CH), jnp.int32),
            pltpu.VMEM((_NDMA, _CH), jnp.float32),
            pltpu.VMEM((_NDMA, _CH), jnp.float32),
            pltpu.VMEM((_RPT,), jnp.float32),
            pltpu.VMEM((_L,), jnp.float32),
            pltpu.SemaphoreType.DMA,
        ],
    )(_sc_body)
    partials = run(pred_flat, idx2, st2, mask2)

    out = pl.pallas_call(
        _combine_body,
        out_shape=jax.ShapeDtypeStruct((1, 1), jnp.float32),
    )(partials)
    return out[0, 0]


# speedup vs baseline: 2.7415x; 2.7415x over previous
import functools

import jax
import jax.numpy as jnp
from jax import lax
from jax.experimental import pallas as pl
from jax.experimental.pallas import tpu as pltpu
from jax.experimental.pallas import tpu_sc as plsc

_L = 16
_NC = 2
_NSC = 16
_NW = _NC * _NSC
_ROWS = 2048
_K = 8
_V = 100000
_E = _ROWS * _K
_EPW = _E // _NW
_RPW = _ROWS // _NW


def _sc_body(pred_hbm, idx_hbm, st_hbm, mask_hbm, out_hbm,
             idx_v, st_v, wgt_v, mask_v, buf0_v, buf1_v, red_v, sem, sem2):
    wid = lax.axis_index("s") * _NC + lax.axis_index("c")
    lane = lax.iota(jnp.int32, _L)
    sub = lane >> 3

    pltpu.sync_copy(idx_hbm.at[wid], idx_v)
    pltpu.sync_copy(st_hbm.at[wid], st_v)
    pltpu.sync_copy(mask_hbm.at[wid], mask_v)

    for j in range(4):
        mrow = mask_v[0, pl.ds(j * _L, _L)]
        for k in range(8):
            seg = pl.ds(k * _L, _L)
            m0 = jnp.full((_L,), mrow[2 * k], jnp.float32)
            m1 = jnp.full((_L,), mrow[2 * k + 1], jnp.float32)
            wgt_v[j, seg] = st_v[j, seg] * jnp.where(sub == 0, m0, m1)

    base_r = wid * _RPW

    def load16(ref, h):
        return ref[h >> 3, pl.ds((h & 7) * _L, _L)]

    def fire(cvec, r, half, buf, sem_p):
        rb = pl.multiple_of((r >> 3) << 3, 8)
        for i in range(8):
            c = cvec[half * 8 + i]
            c128 = pl.multiple_of((c >> 7) << 7, 128)
            pltpu.make_async_copy(
                pred_hbm.at[pl.ds(rb, 8), pl.ds(c128, 128)],
                buf.at[pl.ds(i * 8, 8)], sem_p).start()

    def drain(buf, sem_p):
        for i in range(8):
            pltpu.make_async_copy(
                pred_hbm.at[pl.ds(pl.multiple_of(0, 8), 8),
                            pl.ds(pl.multiple_of(0, 128), 128)],
                buf.at[pl.ds(i * 8, 8)], sem_p).wait()

    def compute(cvec, wvec, r, half, buf, acc):
        sr = r & 7
        for i in range(8):
            c = cvec[half * 8 + i]
            off = ((c >> 4) << 4) - ((c >> 7) << 7)
            win = buf[i * 8 + sr, pl.ds(off, _L)]
            cl = jnp.full((_L,), c & 15, jnp.int32)
            w16 = jnp.full((_L,), wvec[half * 8 + i], jnp.float32)
            acc = acc + jnp.where(lane == cl, win * w16,
                                  jnp.zeros((_L,), jnp.float32))
        return acc

    cvec0 = load16(idx_v, 0)
    fire(cvec0, base_r, 0, buf0_v, sem)

    def body(h, acc):
        cvec = load16(idx_v, h)
        wvec = load16(wgt_v, h)
        r0 = base_r + 2 * h
        fire(cvec, r0 + 1, 1, buf1_v, sem2)
        drain(buf0_v, sem)
        acc = compute(cvec, wvec, r0, 0, buf0_v, acc)

        @pl.when(h < 31)
        def _():
            fire(load16(idx_v, h + 1), r0 + 2, 0, buf0_v, sem)
        drain(buf1_v, sem2)
        return compute(cvec, wvec, r0 + 1, 1, buf1_v, acc)

    acc = lax.fori_loop(0, 32, body, jnp.zeros((_L,), jnp.float32))

    macc = jnp.zeros((_L,), jnp.float32)
    for k in range(4):
        macc = macc + mask_v[0, pl.ds(k * _L, _L)]

    s_t = jnp.sum(acc)
    m_t = jnp.sum(macc)
    partial = jnp.where(lane == 0, jnp.full((_L,), s_t, jnp.float32),
                        jnp.where(lane == 1, jnp.full((_L,), m_t, jnp.float32),
                                  jnp.zeros((_L,), jnp.float32)))
    red_v[0, pl.ds(0, _L)] = partial
    for k in range(1, 8):
        red_v[0, pl.ds(k * _L, _L)] = jnp.zeros((_L,), jnp.float32)
    pltpu.sync_copy(red_v, out_hbm.at[wid])


def _combine_body(p_ref, o_ref):
    p = p_ref[...]
    s = jnp.sum(p[:, 0])
    m = jnp.sum(p[:, 1])
    o_ref[...] = jnp.full((1, 1), -(s / m), jnp.float32)


def kernel(pred, idxs, soft_target, mask):
    pred2 = pred.reshape(_ROWS, _V)
    idx3 = jnp.pad(idxs.astype(jnp.int32).reshape(_NW, 4, 128),
                   ((0, 0), (0, 4), (0, 0)))
    st3 = jnp.pad(soft_target.astype(jnp.float32).reshape(_NW, 4, 128),
                  ((0, 0), (0, 4), (0, 0)))
    mask3 = jnp.pad(mask.astype(jnp.float32).reshape(_NW, 1, _RPW),
                    ((0, 0), (0, 0), (0, 128 - _RPW)))

    mesh = plsc.VectorSubcoreMesh(core_axis_name="c", subcore_axis_name="s")
    run = functools.partial(
        pl.kernel,
        mesh=mesh,
        compiler_params=pltpu.CompilerParams(
            needs_layout_passes=False, use_tc_tiling_on_sc=True),
        out_type=jax.ShapeDtypeStruct((_NW, 1, 128), jnp.float32),
        scratch_types=[
            pltpu.VMEM((8, 128), jnp.int32),
            pltpu.VMEM((8, 128), jnp.float32),
            pltpu.VMEM((8, 128), jnp.float32),
            pltpu.VMEM((1, 128), jnp.float32),
            pltpu.VMEM((64, 128), jnp.float32),
            pltpu.VMEM((64, 128), jnp.float32),
            pltpu.VMEM((1, 128), jnp.float32),
            pltpu.SemaphoreType.DMA,
            pltpu.SemaphoreType.DMA,
        ],
    )(_sc_body)
    partials = run(pred2, idx3, st3, mask3)

    out = pl.pallas_call(
        _combine_body,
        out_shape=jax.ShapeDtypeStruct((1, 1), jnp.float32),
    )(partials.reshape(_NW, 128))
    return out[0, 0]
